# baseline (speedup 1.0000x reference)
import functools

import jax
import jax.numpy as jnp
from jax import lax
from jax.experimental import pallas as pl
from jax.experimental.pallas import tpu as pltpu
from jax.experimental.pallas import tpu_sc as plsc

B, N, C = 64, 8732, 81
NP = 8736
SL = NP // 16
V0, V1 = 0.1, 0.2
TOP_BITS = 0x7F800000


LANE = 128
NFULL = N // LANE
NTAIL = N - NFULL * LANE
CSUB = C // 8


def _tc_body(pconf_ref, ploc_ref, gloc_ref, glabel_ref, dxy_ref, invdwh_ref,
             logdwh_ref, closs_ref, conneg_ref, stats_ref):
    def chunk(col, width):
        sl = pl.ds(col, width)
        g = glabel_ref[0, 0, sl]
        rows = [pconf_ref[0, pl.ds(8 * j, 8), sl] for j in range(CSUB)]
        last = pconf_ref[0, C - 1, sl]
        macc = rows[0]
        for j in range(1, CSUB):
            macc = jnp.maximum(macc, rows[j])
        mx = jnp.maximum(jnp.max(macc, axis=0), last)
        mxb = mx[None, :]
        g8 = g[None, :]
        sacc = jnp.zeros((8, width), jnp.float32)
        selacc = jnp.zeros((8, width), jnp.float32)
        for j in range(CSUB):
            sacc = sacc + jnp.exp(rows[j] - mxb)
            cls = lax.broadcasted_iota(jnp.int32, (8, width), 0) + 8 * j
            selacc = selacc + jnp.where(cls == g8, rows[j], 0.0)
        s = jnp.sum(sacc, axis=0) + jnp.exp(last - mx)
        sel = jnp.sum(selacc, axis=0) + jnp.where(g == C - 1, last, 0.0)
        closs = mx + jnp.log(s) - sel
        mask = g > 0
        maskf = mask.astype(jnp.float32)
        conneg = jnp.where(mask, 0.0, closs)
        closs_ref[0, 0, sl] = closs
        conneg_ref[0, 0, sl] = lax.bitcast_convert_type(conneg, jnp.int32)
        return closs * maskf, maskf

    def body(t, carry):
        pv, nv = carry
        cp, mf = chunk(t * LANE, LANE)
        return pv + cp, nv + mf

    pv, nv = lax.fori_loop(
        0, NFULL, body,
        (jnp.zeros((LANE,), jnp.float32), jnp.zeros((LANE,), jnp.float32)))
    ct, mt = chunk(NFULL * LANE, NTAIL)
    pos_sum = jnp.sum(pv) + jnp.sum(ct)
    npos = jnp.sum(nv) + jnp.sum(mt)

    glabel = glabel_ref[0, 0]
    maskf_all = (glabel > 0).astype(jnp.float32)
    ploc = ploc_ref[0]
    gloc = gloc_ref[0]
    gxy = (gloc[:2] - dxy_ref[0]) * invdwh_ref[0]
    gwh = (jnp.log(gloc[2:]) - logdwh_ref[0]) * (1.0 / V1)
    g = jnp.concatenate([gxy, gwh], axis=0)
    d = ploc - g
    ad = jnp.abs(d)
    sl1 = jnp.sum(jnp.where(ad < 1.0, 0.5 * d * d, ad - 0.5), axis=0)
    loc_loss = jnp.sum(sl1 * maskf_all)
    stats_ref[0, 0] = jnp.stack([loc_loss, pos_sum, npos])


def _tc_stage(pconf, ploc, gloc, glabel3, dxy, invdwh, logdwh):
    return pl.pallas_call(
        _tc_body,
        grid=(B,),
        in_specs=[
            pl.BlockSpec((1, C, N), lambda b: (b, 0, 0)),
            pl.BlockSpec((1, 4, N), lambda b: (b, 0, 0)),
            pl.BlockSpec((1, 4, N), lambda b: (b, 0, 0)),
            pl.BlockSpec((1, 1, N), lambda b: (b, 0, 0)),
            pl.BlockSpec((1, 2, N), lambda b: (0, 0, 0)),
            pl.BlockSpec((1, 2, N), lambda b: (0, 0, 0)),
            pl.BlockSpec((1, 2, N), lambda b: (0, 0, 0)),
        ],
        out_specs=[
            pl.BlockSpec((1, 1, N), lambda b: (b, 0, 0)),
            pl.BlockSpec((1, 1, N), lambda b: (b, 0, 0)),
            pl.BlockSpec((1, 1, 3), lambda b: (b, 0, 0)),
        ],
        out_shape=[
            jax.ShapeDtypeStruct((B, 1, N), jnp.float32),
            jax.ShapeDtypeStruct((B, 1, N), jnp.int32),
            jax.ShapeDtypeStruct((B, 1, 3), jnp.float32),
        ],
    )(pconf, ploc, gloc, glabel3, dxy, invdwh, logdwh)


def _sc_mining(conneg_hbm, closs_hbm, k_hbm, out_hbm, cn_v, cl_v, k_v, o_v):
    cid = lax.axis_index("c")
    sid = lax.axis_index("s")
    wid = sid * 2 + cid
    zi = jnp.zeros((16,), jnp.int32)
    zf = jnp.zeros((16,), jnp.float32)
    oi = jnp.full((16,), 1, jnp.int32)

    for r in range(2):
        row = wid * 2 + r
        pltpu.sync_copy(conneg_hbm.at[row], cn_v.at[r])
        pltpu.sync_copy(closs_hbm.at[row], cl_v.at[r])
        pltpu.sync_copy(k_hbm.at[row], k_v)
        kk = k_v[...]

        def bits_at(i):
            return cn_v[r, pl.ds(i * 16, 16)]

        def gt_pass(vkb):
            def gt_body(i, st):
                cg, sg = st
                gt = bits_at(i) > vkb
                c = cl_v[r, pl.ds(i * 16, 16)]
                return cg + jnp.where(gt, oi, zi), sg + jnp.where(gt, c, zf)

            cgv, sgv = lax.fori_loop(0, SL, gt_body, (zi, zf))
            return jnp.sum(cgv), jnp.sum(sgv)

        cnt0, sum0 = gt_pass(zi)

        def slow(_):
            def bis(_, st):
                lo, hi = st
                mid = lo + lax.shift_right_logical(hi - lo, 1)

                def cnt_body(i, acc):
                    return acc + jnp.where(bits_at(i) >= mid, oi, zi)

                cnt = jnp.sum(lax.fori_loop(0, SL, cnt_body, zi))
                pred = jnp.full((16,), cnt, jnp.int32) >= kk
                return jnp.where(pred, mid, lo), jnp.where(pred, hi, mid)

            vkb, _ = lax.fori_loop(
                0, 31, bis, (jnp.full((16,), 1, jnp.int32),
                             jnp.full((16,), TOP_BITS, jnp.int32)))
            cg, sg = gt_pass(vkb)
            return vkb, cg, sg

        vkb, count_gt, sum_gt = lax.cond(
            cnt0 < jnp.max(kk),
            lambda _: (zi, cnt0, sum0), slow, 0)
        m = kk - jnp.full((16,), count_gt, jnp.int32)

        def tie_body(i, st):
            carry, acc = st
            tie = bits_at(i) == vkb
            pref = plsc.cumsum(jnp.where(tie, oi, zi)) + carry
            sel = tie & (pref <= m)
            c = cl_v[r, pl.ds(i * 16, 16)]
            acc = acc + jnp.where(sel, c, zf)
            carry = carry + plsc.all_reduce_population_count(tie)
            return carry, acc

        _, accv = lax.fori_loop(0, SL, tie_body, (zi, zf))
        o_v[...] = jnp.full((16,), sum_gt + jnp.sum(accv), jnp.float32)
        pltpu.sync_copy(o_v, out_hbm.at[row])


def _sc_stage(conneg, closs, kvec):
    mesh = plsc.VectorSubcoreMesh(core_axis_name="c", subcore_axis_name="s")
    return pl.kernel(
        _sc_mining,
        out_type=jax.ShapeDtypeStruct((B, 16), jnp.float32),
        mesh=mesh,
        compiler_params=pltpu.CompilerParams(needs_layout_passes=False),
        scratch_types=[
            pltpu.VMEM((2, NP), jnp.int32),
            pltpu.VMEM((2, NP), jnp.float32),
            pltpu.VMEM((16,), jnp.int32),
            pltpu.VMEM((16,), jnp.float32),
        ],
    )(conneg, closs, kvec)


@jax.jit
def kernel(ploc, pconf, gloc, glabel, dboxes):
    dxy = dboxes[:, :2, :]
    dwh = dboxes[:, 2:, :]
    invdwh = 1.0 / (V0 * dwh)
    logdwh = jnp.log(dwh)
    glabel3 = glabel[:, None, :]

    closs, conneg, stats = _tc_stage(
        pconf, ploc, gloc, glabel3, dxy, invdwh, logdwh)
    closs = closs[:, 0, :]
    conneg = conneg[:, 0, :]
    loc_loss = stats[:, 0, 0]
    pos_sum = stats[:, 0, 1]
    num_pos = stats[:, 0, 2]

    pad = ((0, 0), (0, NP - N))
    closs_p = jnp.pad(closs, pad)
    conneg_p = jnp.pad(conneg, pad)
    k = jnp.minimum(3 * num_pos.astype(jnp.int32), N)
    kvec = jnp.broadcast_to(k[:, None], (B, 16)).astype(jnp.int32)

    neg_sum = _sc_stage(conneg_p, closs_p, kvec)[:, 0]

    total = loc_loss + pos_sum + neg_sum
    num_mask = (num_pos > 0).astype(jnp.float32)
    return (total * num_mask / jnp.maximum(num_pos, 1e-6)).mean(axis=0)
